# baseline (speedup 1.0000x reference)
import functools

import jax
import jax.numpy as jnp
from jax import lax
from jax.experimental import pallas as pl
from jax.experimental.pallas import tpu as pltpu
from jax.experimental.pallas import tpu_sc as plsc

_THRESHOLD = 0.8
_N = 5000
_NP = 5120
_SC_ROWS = 1024
_TC_ROWS = _N - _SC_ROWS
_TC_BLOCK = 568
_NW = 32
_RPW = _SC_ROWS // _NW


def _tc_kernel(bi_ref, bjt_ref, boxes_out_ref, keep_out_ref):
    bi = bi_ref[...]
    x1i, y1i, x2i, y2i = (bi[:, 0:1], bi[:, 1:2], bi[:, 2:3], bi[:, 3:4])
    x1j = bjt_ref[0:1, :]
    y1j = bjt_ref[1:2, :]
    x2j = bjt_ref[2:3, :]
    y2j = bjt_ref[3:4, :]
    area_j = (x2j - x1j) * (y2j - y1j)
    contained = (
        (x1j >= x1i) & (y1j >= y1i) & (x2j <= x2i) & (y2j <= y2i)
    )
    s = jnp.sum(
        jnp.where(contained, jnp.broadcast_to(area_j, contained.shape), 0.0),
        axis=1,
        keepdims=True,
    )
    area_i = (x2i - x1i) * (y2i - y1i)
    s = s - area_i
    keep = jnp.where(s <= _THRESHOLD * (area_i + 1e-9), 1.0, 0.0)
    keep_out_ref[...] = keep
    boxes_out_ref[...] = bi * keep


def _tc_call(boxes, bt):
    grid = _TC_ROWS // _TC_BLOCK
    return pl.pallas_call(
        _tc_kernel,
        grid=(grid,),
        in_specs=[
            pl.BlockSpec((_TC_BLOCK, 4), lambda i: (i, 0)),
            pl.BlockSpec((4, _NP), lambda i: (0, 0)),
        ],
        out_specs=[
            pl.BlockSpec((_TC_BLOCK, 4), lambda i: (i, 0)),
            pl.BlockSpec((_TC_BLOCK, 1), lambda i: (i, 0)),
        ],
        out_shape=[
            jax.ShapeDtypeStruct((_TC_ROWS, 4), boxes.dtype),
            jax.ShapeDtypeStruct((_TC_ROWS, 1), jnp.float32),
        ],
        compiler_params=pltpu.CompilerParams(
            dimension_semantics=("parallel",)
        ),
    )(boxes[:_TC_ROWS], bt)


@functools.partial(
    pl.kernel,
    mesh=plsc.VectorSubcoreMesh(core_axis_name="c", subcore_axis_name="s"),
    out_type=jax.ShapeDtypeStruct((_SC_ROWS,), jnp.float32),
    scratch_types=[
        pltpu.VMEM((5, _NP), jnp.float32),
        pltpu.VMEM((5, _RPW * 16), jnp.float32),
        pltpu.VMEM((_RPW,), jnp.float32),
        pltpu.VMEM((32,), jnp.float32),
    ],
)
def _sc_kernel(jall_h, irep_h, out_h, jv, iv, kv, rbuf):
    wid = lax.axis_index("s") * 2 + lax.axis_index("c")
    pltpu.sync_copy(jall_h, jv)
    pltpu.sync_copy(irep_h.at[:, pl.ds(wid * _RPW * 16, _RPW * 16)], iv)
    lane = lax.iota(jnp.int32, 16)

    def blk_body(b, _):
        def row_body(q, keep_vec):
            r = b * 16 + q
            rs = pl.ds(r * 16, 16)
            x1s = iv[0, rs]
            y1s = iv[1, rs]
            x2s = iv[2, rs]
            y2s = iv[3, rs]
            areas = iv[4, rs]

            def j_body(jj, acc):
                for u in range(8):
                    sl = pl.ds((jj * 8 + u) * 16, 16)
                    m = (
                        (jv[0, sl] >= x1s)
                        & (jv[1, sl] >= y1s)
                        & (jv[2, sl] <= x2s)
                        & (jv[3, sl] <= y2s)
                    )
                    acc = acc + jnp.where(m, jv[4, sl], 0.0)
                return acc

            acc = lax.fori_loop(
                0, _NP // 128, j_body, jnp.zeros((16,), jnp.float32)
            )
            for sh in (8, 4, 2, 1):
                rbuf[pl.ds(0, 16)] = acc
                rbuf[pl.ds(16, 16)] = acc
                acc = acc + rbuf[pl.ds(sh, 16)]
            s = acc - areas
            keep = jnp.where(
                s <= _THRESHOLD * (areas + 1e-9),
                jnp.ones((16,), jnp.float32),
                jnp.zeros((16,), jnp.float32),
            )
            return jnp.where(lane == q, keep, keep_vec)

        keep_vec = lax.fori_loop(
            0, 16, row_body, jnp.zeros((16,), jnp.float32)
        )
        kv[pl.ds(b * 16, 16)] = keep_vec
        return 0

    lax.fori_loop(0, _RPW // 16, blk_body, 0)
    pltpu.sync_copy(kv, out_h.at[pl.ds(wid * _RPW, _RPW)])


def kernel(boxes):
    n = boxes.shape[0]
    bt = jnp.zeros((4, _NP), boxes.dtype).at[:, :n].set(boxes.T)
    area_row = (bt[2] - bt[0]) * (bt[3] - bt[1])
    jall = jnp.concatenate([bt, area_row[None]], axis=0)
    irep = jnp.repeat(jall[:, _TC_ROWS:n], 16, axis=1)
    keep_sc = _sc_kernel(jall, irep)
    boxes_tc, keep_tc = _tc_call(boxes, bt)
    keep = jnp.concatenate([keep_tc[:, 0], keep_sc]) > 0.5
    boxes_out = jnp.concatenate(
        [boxes_tc, boxes[_TC_ROWS:] * keep_sc[:, None]], axis=0
    )
    return boxes_out, keep


# speedup vs baseline: 1.2719x; 1.2719x over previous
import jax
import jax.numpy as jnp
from jax.experimental import pallas as pl
from jax.experimental.pallas import tpu as pltpu

_THRESHOLD = 0.8
_ROWS = 1000
_LANE_PAD = 128


def _prior_filter_kernel(bi_ref, bjt_ref, boxes_out_ref, keep_out_ref):
    bi = bi_ref[...]
    x1i, y1i, x2i, y2i = (bi[:, 0:1], bi[:, 1:2], bi[:, 2:3], bi[:, 3:4])
    x1j = bjt_ref[0:1, :]
    y1j = bjt_ref[1:2, :]
    x2j = bjt_ref[2:3, :]
    y2j = bjt_ref[3:4, :]
    area_j = (x2j - x1j) * (y2j - y1j)
    contained = (
        (x1j >= x1i) & (y1j >= y1i) & (x2j <= x2i) & (y2j <= y2i)
    )
    s = jnp.sum(
        jnp.where(contained, jnp.broadcast_to(area_j, contained.shape), 0.0),
        axis=1,
        keepdims=True,
    )
    area_i = (x2i - x1i) * (y2i - y1i)
    s = s - area_i
    keep = jnp.where(
        s <= _THRESHOLD * (area_i + 1e-9), 1.0, 0.0
    )
    keep_out_ref[...] = keep
    boxes_out_ref[...] = bi * keep


def kernel(boxes):
    n = boxes.shape[0]
    npad = ((n + _LANE_PAD - 1) // _LANE_PAD) * _LANE_PAD
    bt = jnp.zeros((4, npad), boxes.dtype).at[:, :n].set(boxes.T)
    grid = n // _ROWS
    boxes_out, keep = pl.pallas_call(
        _prior_filter_kernel,
        grid=(grid,),
        in_specs=[
            pl.BlockSpec((_ROWS, 4), lambda i: (i, 0)),
            pl.BlockSpec((4, npad), lambda i: (0, 0)),
        ],
        out_specs=[
            pl.BlockSpec((_ROWS, 4), lambda i: (i, 0)),
            pl.BlockSpec((_ROWS, 1), lambda i: (i, 0)),
        ],
        out_shape=[
            jax.ShapeDtypeStruct((n, 4), boxes.dtype),
            jax.ShapeDtypeStruct((n, 1), jnp.float32),
        ],
        compiler_params=pltpu.CompilerParams(
            dimension_semantics=("parallel",)
        ),
    )(boxes, bt)
    return boxes_out, keep[:, 0] > 0.5


# speedup vs baseline: 1.2725x; 1.0005x over previous
import jax
import jax.numpy as jnp
from jax.experimental import pallas as pl
from jax.experimental.pallas import tpu as pltpu

_THRESHOLD = 0.8
_ROWS = 1000
_LANE_PAD = 128


def _prior_filter_kernel(bi_ref, bjt_ref, boxes_out_ref, keep_out_ref):
    bi = bi_ref[...]
    x1i, y1i, x2i, y2i = (bi[:, 0:1], bi[:, 1:2], bi[:, 2:3], bi[:, 3:4])
    x1j = bjt_ref[0:1, :]
    y1j = bjt_ref[1:2, :]
    x2j = bjt_ref[2:3, :]
    y2j = bjt_ref[3:4, :]
    area_j = (x2j - x1j) * (y2j - y1j)
    contained = (
        (x1j >= x1i) & (y1j >= y1i) & (x2j <= x2i) & (y2j <= y2i)
    )
    s = jnp.sum(
        jnp.where(contained, jnp.broadcast_to(area_j, contained.shape), 0.0),
        axis=1,
        keepdims=True,
    )
    area_i = (x2i - x1i) * (y2i - y1i)
    s = s - area_i
    keep = jnp.where(
        s <= _THRESHOLD * (area_i + 1e-9), 1.0, 0.0
    )
    keep_out_ref[...] = keep
    boxes_out_ref[...] = bi * keep


def kernel(boxes):
    n = boxes.shape[0]
    npad = ((n + _LANE_PAD - 1) // _LANE_PAD) * _LANE_PAD
    bt = jnp.zeros((4, npad), boxes.dtype).at[:, :n].set(boxes.T)
    grid = n // _ROWS
    boxes_out, keep = pl.pallas_call(
        _prior_filter_kernel,
        grid=(grid,),
        in_specs=[
            pl.BlockSpec((_ROWS, 4), lambda i: (i, 0)),
            pl.BlockSpec((4, npad), lambda i: (0, 0)),
        ],
        out_specs=[
            pl.BlockSpec((_ROWS, 4), lambda i: (i, 0)),
            pl.BlockSpec((_ROWS, 1), lambda i: (i, 0)),
        ],
        out_shape=[
            jax.ShapeDtypeStruct((n, 4), boxes.dtype),
            jax.ShapeDtypeStruct((n, 1), jnp.float32),
        ],
        compiler_params=pltpu.CompilerParams(
            dimension_semantics=("arbitrary",)
        ),
    )(boxes, bt)
    return boxes_out, keep[:, 0] > 0.5
